# baseline (speedup 1.0000x reference)
import jax
import jax.numpy as jnp
from jax.experimental import pallas as pl
from jax.experimental.pallas import tpu as pltpu

_N = 200
_NB = 64
_BB = 64
_G = 8


def _pb_kernel(posw_ref, out_ref):
    for i in range(_N):
        out_ref[i, :] = posw_ref[0, _N - 1 - i : 2 * _N - 1 - i]


def _bucket_val(diff, table, rows):
    mag = jnp.maximum(jnp.abs(diff.astype(jnp.float32)), 1.0)
    bk = (jnp.log(mag) / 0.301).astype(jnp.int32)
    cols = diff.shape[-1]
    g = jnp.take_along_axis(
        table[: _G * rows], bk.reshape(_G * rows, cols), axis=-1,
        mode="promise_in_bounds",
    )
    return g.reshape(_G, rows, cols)


def _main_kernel(ts_ref, tsw_ref, pb_ref, out_ref):
    ts = ts_ref[...]
    shifted = jnp.concatenate([ts[:, 1:], ts[:, _N - 1 : _N]], axis=1)
    pbv = pb_ref[...]
    pbA = pbv[None, :, :128]
    pbB = pbv[None, 120:, 128:]
    pbR = pbv[None, :120, 128:]
    table = jnp.broadcast_to(tsw_ref[0, :], (_G * _N, 128))
    for b0 in range(0, _BB, _G):
        tsg = ts[b0 : b0 + _G]
        shg = shifted[b0 : b0 + _G, :, None]
        A = _bucket_val(shg - tsg[:, None, :128], table, _N)
        B = _bucket_val(shg[:, 120:] - tsg[:, None, 128:], table, 80)
        R = jnp.swapaxes(A[:, 127:199, 1:121], 1, 2)
        out_ref[b0 : b0 + _G, :, :128] = A + pbA
        out_ref[b0 : b0 + _G, 120:, 128:] = B + pbB
        out_ref[b0 : b0 + _G, :120, 128:] = R + pbR


def kernel(ts, ts_w, pos_w):
    B, N = ts.shape
    posw_pad = jnp.zeros((1, 512), jnp.float32).at[0, : 2 * N - 1].set(pos_w)
    pb = pl.pallas_call(
        _pb_kernel,
        out_shape=jax.ShapeDtypeStruct((N, N), jnp.float32),
    )(posw_pad)

    tsw_pad = jnp.full((1, 128), ts_w[_NB], jnp.float32).at[0, : _NB + 1].set(ts_w)
    return pl.pallas_call(
        _main_kernel,
        grid=(B // _BB,),
        in_specs=[
            pl.BlockSpec((_BB, N), lambda b: (b, 0)),
            pl.BlockSpec((1, 128), lambda b: (0, 0)),
            pl.BlockSpec((N, N), lambda b: (0, 0)),
        ],
        out_specs=pl.BlockSpec((_BB, N, N), lambda b: (b, 0, 0)),
        out_shape=jax.ShapeDtypeStruct((B, N, N), jnp.float32),
        compiler_params=pltpu.CompilerParams(
            dimension_semantics=("parallel",),
        ),
    )(ts, tsw_pad, pb)
